# baseline (speedup 1.0000x reference)
import functools

import jax
import jax.numpy as jnp
from jax import lax
from jax.experimental import pallas as pl
from jax.experimental.pallas import tpu as pltpu
from jax.experimental.pallas import tpu_sc as plsc

NUM_CORES = 2
NUM_SUBCORES = 16
NW = NUM_CORES * NUM_SUBCORES
LANES = 16
CHUNK = 64
NBUF = 4
NOUT = 2


def _gru_body(d, parts_ref, h_ref, wihT_ref, whhT_ref, bih_ref, bhh_ref, h_out_ref):
    agg = parts_ref[0] + parts_ref[1]
    gi = jnp.dot(agg, wihT_ref[...], preferred_element_type=jnp.float32) + bih_ref[...]
    gh = jnp.dot(h_ref[...], whhT_ref[...], preferred_element_type=jnp.float32) + bhh_ref[...]
    r = jax.nn.sigmoid(gi[:, 0:d] + gh[:, 0:d])
    z = jax.nn.sigmoid(gi[:, d:2 * d] + gh[:, d:2 * d])
    ng = jnp.tanh(gi[:, 2 * d:] + r * gh[:, 2 * d:])
    h_out_ref[...] = (1.0 - z) * ng + z * h_ref[...]


def _tc_gru(parts, h, wihT, whhT, bih, bhh):
    n, d = h.shape
    blk = 1000 if n % 1000 == 0 else n
    grid = n // blk
    return pl.pallas_call(
        functools.partial(_gru_body, d),
        grid=(grid,),
        in_specs=[pl.BlockSpec((NUM_CORES, blk, d), lambda i: (0, i, 0)),
                  pl.BlockSpec((blk, d), lambda i: (i, 0)),
                  pl.BlockSpec((d, 3 * d), lambda i: (0, 0)),
                  pl.BlockSpec((d, 3 * d), lambda i: (0, 0)),
                  pl.BlockSpec((1, 3 * d), lambda i: (0, 0)),
                  pl.BlockSpec((1, 3 * d), lambda i: (0, 0))],
        out_specs=pl.BlockSpec((blk, d), lambda i: (i, 0)),
        out_shape=jax.ShapeDtypeStruct((n, d), jnp.float32),
    )(parts, h, wihT, whhT, bih, bhh)


def _bcast_lane(vec16, j):
    idx = jnp.full((LANES, 1), j, dtype=jnp.int32)
    dnums = lax.GatherDimensionNumbers(
        offset_dims=(), collapsed_slice_dims=(0,), start_index_map=(0,))
    return lax.gather(vec16, idx, dnums, (1,),
                      mode=lax.GatherScatterMode.PROMISE_IN_BOUNDS)


def _sc_edge_agg(m, src, dst, ew, nchunk):
    n, d = m.shape
    rps = (n // NUM_SUBCORES) // 8 * 8
    rem = n - NUM_SUBCORES * rps
    mesh = plsc.VectorSubcoreMesh(core_axis_name="c", subcore_axis_name="s")

    @functools.partial(
        pl.kernel,
        out_type=jax.ShapeDtypeStruct((NUM_CORES, n, d), jnp.float32),
        mesh=mesh,
        scratch_types=[
            [pltpu.VMEM((CHUNK,), jnp.int32)] * NBUF,
            [pltpu.VMEM((CHUNK,), jnp.int32)] * NBUF,
            [pltpu.VMEM((CHUNK,), jnp.float32)] * NBUF,
            [pltpu.VMEM((CHUNK,), jnp.int32)] * NOUT,
            [pltpu.VMEM((CHUNK, d), jnp.float32)] * NBUF,
            [pltpu.VMEM((CHUNK, d), jnp.float32)] * NOUT,
            pltpu.VMEM_SHARED((n, d), jnp.float32),
            [pltpu.SemaphoreType.DMA] * NBUF,
            [pltpu.SemaphoreType.DMA] * NOUT,
            [pltpu.SemaphoreType.DMA] * NBUF,
        ],
    )
    def k(m_hbm, src_hbm, dst_hbm, w_hbm, out_hbm,
          srcb, dstb, wb_, dbufs, rowbufs, outbufs, acc_sh, gsems, ssems, isems):
        cid = lax.axis_index("c")
        sid = lax.axis_index("s")
        wid = cid * NUM_SUBCORES + sid

        def idx_copies(b, i):
            return (pltpu.make_async_copy(src_hbm.at[wid, i], srcb[b], isems[b]),
                    pltpu.make_async_copy(dst_hbm.at[wid, i], dstb[b], isems[b]),
                    pltpu.make_async_copy(w_hbm.at[wid, i], wb_[b], isems[b]))

        def idxload(b, i):
            for c in idx_copies(b, i):
                c.start()

        def iwait(b, i):
            for c in idx_copies(b, i):
                c.wait()

        def scale(b, o, i):
            rows, out, dbuf, wv, dv = (rowbufs[b], outbufs[o], dbufs[o],
                                       wb_[b], dstb[b])

            @pl.loop(0, CHUNK // LANES)
            def _(g):
                w16 = wv[pl.ds(g * LANES, LANES)]
                dbuf[pl.ds(g * LANES, LANES)] = dv[pl.ds(g * LANES, LANES)]
                for j in range(LANES):
                    wbc = _bcast_lane(w16, j)
                    e = g * LANES + j
                    for c in range(d // LANES):
                        sl = (e, pl.ds(c * LANES, LANES))
                        out[sl] = rows[sl] * wbc

        def gather(b):
            pltpu.async_copy(m_hbm.at[srcb[b]], rowbufs[b], gsems[b])

        def gwait(b):
            pltpu.make_async_copy(m_hbm.at[srcb[b]], rowbufs[b],
                                  gsems[b]).wait()

        def scatter(o):
            pltpu.async_copy(outbufs[o], acc_sh.at[dbufs[o]], ssems[o],
                             add=True)

        def swait(o):
            pltpu.make_async_copy(outbufs[o], acc_sh.at[dbufs[o]],
                                  ssems[o]).wait()

        for b in range(NBUF):
            idxload(b, b)

        rows0 = outbufs[0]

        @pl.loop(0, CHUNK)
        def _(r):
            for c in range(d // LANES):
                rows0[r, pl.ds(c * LANES, LANES)] = (
                    jnp.zeros((LANES,), jnp.float32))
        row0 = sid * rps
        for off in range(0, rps, CHUNK):
            sz = min(CHUNK, rps - off)
            pltpu.sync_copy(rows0.at[pl.ds(0, sz)],
                            acc_sh.at[pl.ds(row0 + off, sz)])
        if rem:
            @pl.when(sid == NUM_SUBCORES - 1)
            def _():
                for off in range(0, rem, CHUNK):
                    sz = min(CHUNK, rem - off)
                    pltpu.sync_copy(rows0.at[pl.ds(0, sz)],
                                    acc_sh.at[pl.ds(NUM_SUBCORES * rps + off, sz)])
        plsc.subcore_barrier()

        iwait(0, 0)
        gather(0)
        iwait(1, 1)
        gather(1)

        @pl.loop(0, nchunk // NBUF)
        def _(t):
            i0 = t * NBUF
            for b in range(NBUF):
                i = i0 + b
                o = b % NOUT
                gwait(b)
                if b < NOUT:
                    @pl.when(t > 0)
                    def _():
                        swait(o)
                else:
                    swait(o)
                scale(b, o, i)

                @pl.when(i + NBUF < nchunk)
                def _():
                    idxload(b, i + NBUF)
                scatter(o)
                q = (b + 2) % NBUF
                if b < 2:
                    iwait(q, i + 2)
                    gather(q)
                else:
                    @pl.when(i + 2 < nchunk)
                    def _():
                        iwait(q, i + 2)
                        gather(q)

        swait(0)
        swait(1)
        plsc.subcore_barrier()

        for off in range(0, rps, CHUNK):
            sz = min(CHUNK, rps - off)
            pltpu.sync_copy(acc_sh.at[pl.ds(row0 + off, sz)],
                            out_hbm.at[cid, pl.ds(row0 + off, sz)])
        if rem:
            @pl.when(sid == NUM_SUBCORES - 1)
            def _():
                for off in range(0, rem, CHUNK):
                    sz = min(CHUNK, rem - off)
                    r0 = NUM_SUBCORES * rps + off
                    pltpu.sync_copy(acc_sh.at[pl.ds(r0, sz)],
                                    out_hbm.at[cid, pl.ds(r0, sz)])

    return k(m, src, dst, ew)


def kernel(x, edge_index, edge_weight, weight, w_ih, w_hh, b_ih, b_hh):
    n, d = x.shape
    e = edge_index.shape[1]
    nchunk = -(-e // (NW * CHUNK))
    nchunk = -(-nchunk // 4) * 4
    e_pad = NW * nchunk * CHUNK

    src = edge_index[0].astype(jnp.int32)
    dst = edge_index[1].astype(jnp.int32)
    ew = edge_weight.astype(jnp.float32)
    pad = e_pad - e
    if pad:
        spread = jnp.arange(pad, dtype=jnp.int32) % n
        src = jnp.concatenate([src, spread])
        dst = jnp.concatenate([dst, spread])
        ew = jnp.concatenate([ew, jnp.zeros((pad,), jnp.float32)])
    src = src.reshape(NW, nchunk, CHUNK)
    dst = dst.reshape(NW, nchunk, CHUNK)
    ew = ew.reshape(NW, nchunk, CHUNK)

    wihT = w_ih.T
    cwihT = [lax.dot(weight[i], wihT, precision=lax.Precision.HIGHEST)
             for i in range(weight.shape[0])]
    whhT = w_hh.T
    bih = b_ih.reshape(1, -1)
    bhh = b_hh.reshape(1, -1)

    h = x
    for i in range(weight.shape[0]):
        parts = _sc_edge_agg(h, src, dst, ew, nchunk)
        h = _tc_gru(parts, h, cwihT[i], whhT, bih, bhh)
    return h


# speedup vs baseline: 1.0297x; 1.0297x over previous
import functools

import jax
import jax.numpy as jnp
from jax import lax
from jax.experimental import pallas as pl
from jax.experimental.pallas import tpu as pltpu
from jax.experimental.pallas import tpu_sc as plsc

NUM_CORES = 2
NUM_SUBCORES = 16
NW = NUM_CORES * NUM_SUBCORES
LANES = 16
CHUNK = 80
NBUF = 4


def _gh_body(x_ref, w_ref, b_ref, o_ref):
    o_ref[...] = jnp.dot(x_ref[...], w_ref[...],
                         preferred_element_type=jnp.float32) + b_ref[...]


def _tc_gh(h, whhT, bhh):
    n, d = h.shape
    blk = 1000 if n % 1000 == 0 else n
    grid = n // blk
    return pl.pallas_call(
        _gh_body,
        grid=(grid,),
        in_specs=[pl.BlockSpec((blk, d), lambda i: (i, 0)),
                  pl.BlockSpec((d, 3 * d), lambda i: (0, 0)),
                  pl.BlockSpec((1, 3 * d), lambda i: (0, 0))],
        out_specs=pl.BlockSpec((blk, 3 * d), lambda i: (i, 0)),
        out_shape=jax.ShapeDtypeStruct((n, 3 * d), jnp.float32),
    )(h, whhT, bhh)


def _gru_body(d, parts_ref, h_ref, gh_ref, wihT_ref, bih_ref, h_out_ref):
    agg = parts_ref[0] + parts_ref[1]
    gi = jnp.dot(agg, wihT_ref[...], preferred_element_type=jnp.float32) + bih_ref[...]
    gh = gh_ref[...]
    r = jax.nn.sigmoid(gi[:, 0:d] + gh[:, 0:d])
    z = jax.nn.sigmoid(gi[:, d:2 * d] + gh[:, d:2 * d])
    ng = jnp.tanh(gi[:, 2 * d:] + r * gh[:, 2 * d:])
    h_out_ref[...] = (1.0 - z) * ng + z * h_ref[...]


def _tc_gru(parts, h, gh, wihT, bih):
    n, d = h.shape
    blk = 1000 if n % 1000 == 0 else n
    grid = n // blk
    return pl.pallas_call(
        functools.partial(_gru_body, d),
        grid=(grid,),
        in_specs=[pl.BlockSpec((NUM_CORES, blk, d), lambda i: (0, i, 0)),
                  pl.BlockSpec((blk, d), lambda i: (i, 0)),
                  pl.BlockSpec((blk, 3 * d), lambda i: (i, 0)),
                  pl.BlockSpec((d, 3 * d), lambda i: (0, 0)),
                  pl.BlockSpec((1, 3 * d), lambda i: (0, 0))],
        out_specs=pl.BlockSpec((blk, d), lambda i: (i, 0)),
        out_shape=jax.ShapeDtypeStruct((n, d), jnp.float32),
    )(parts, h, gh, wihT, bih)


def _bcast_lane(vec16, j):
    idx = jnp.full((LANES, 1), j, dtype=jnp.int32)
    dnums = lax.GatherDimensionNumbers(
        offset_dims=(), collapsed_slice_dims=(0,), start_index_map=(0,))
    return lax.gather(vec16, idx, dnums, (1,),
                      mode=lax.GatherScatterMode.PROMISE_IN_BOUNDS)


def _sc_edge_agg(m, src, dst, ew, nchunk):
    n, d = m.shape
    rps = (n // NUM_SUBCORES) // 8 * 8
    rem = n - NUM_SUBCORES * rps
    mesh = plsc.VectorSubcoreMesh(core_axis_name="c", subcore_axis_name="s")

    @functools.partial(
        pl.kernel,
        out_type=jax.ShapeDtypeStruct((NUM_CORES, n, d), jnp.float32),
        mesh=mesh,
        scratch_types=[
            [pltpu.VMEM((CHUNK,), jnp.int32)] * NBUF,
            [pltpu.VMEM((CHUNK,), jnp.int32)] * NBUF,
            [pltpu.VMEM((CHUNK,), jnp.float32)] * NBUF,
            [pltpu.VMEM((CHUNK,), jnp.int32)] * NBUF,
            [pltpu.VMEM((CHUNK, d), jnp.float32)] * NBUF,
            pltpu.VMEM_SHARED((n, d), jnp.float32),
            [pltpu.SemaphoreType.DMA] * NBUF,
            [pltpu.SemaphoreType.DMA] * NBUF,
            [pltpu.SemaphoreType.DMA] * NBUF,
        ],
    )
    def k(m_hbm, src_hbm, dst_hbm, w_hbm, out_hbm,
          srcb, dstb, wb_, dbufs, rowbufs, acc_sh, gsems, ssems, isems):
        cid = lax.axis_index("c")
        sid = lax.axis_index("s")
        wid = cid * NUM_SUBCORES + sid

        def idx_copies(b, i):
            return (pltpu.make_async_copy(src_hbm.at[wid, i], srcb[b], isems[b]),
                    pltpu.make_async_copy(dst_hbm.at[wid, i], dstb[b], isems[b]),
                    pltpu.make_async_copy(w_hbm.at[wid, i], wb_[b], isems[b]))

        def idxload(b, i):
            for c in idx_copies(b, i):
                c.start()

        def iwait(b, i):
            for c in idx_copies(b, i):
                c.wait()

        def scale(b, i):
            rows, dbuf, wv, dv = rowbufs[b], dbufs[b], wb_[b], dstb[b]

            @pl.loop(0, CHUNK // LANES)
            def _(g):
                w16 = wv[pl.ds(g * LANES, LANES)]
                dbuf[pl.ds(g * LANES, LANES)] = dv[pl.ds(g * LANES, LANES)]
                for j in range(LANES):
                    wbc = _bcast_lane(w16, j)
                    e = g * LANES + j
                    for c in range(d // LANES):
                        sl = (e, pl.ds(c * LANES, LANES))
                        rows[sl] = rows[sl] * wbc

        def gather(b):
            pltpu.async_copy(m_hbm.at[srcb[b]], rowbufs[b], gsems[b])

        def gwait(b):
            pltpu.make_async_copy(m_hbm.at[srcb[b]], rowbufs[b],
                                  gsems[b]).wait()

        def scatter(b):
            pltpu.async_copy(rowbufs[b], acc_sh.at[dbufs[b]], ssems[b],
                             add=True)

        def swait(b):
            pltpu.make_async_copy(rowbufs[b], acc_sh.at[dbufs[b]],
                                  ssems[b]).wait()

        for b in range(NBUF):
            idxload(b, b)

        rows0 = rowbufs[0]

        @pl.loop(0, CHUNK)
        def _(r):
            for c in range(d // LANES):
                rows0[r, pl.ds(c * LANES, LANES)] = (
                    jnp.zeros((LANES,), jnp.float32))
        row0 = sid * rps
        for off in range(0, rps, CHUNK):
            sz = min(CHUNK, rps - off)
            pltpu.sync_copy(rows0.at[pl.ds(0, sz)],
                            acc_sh.at[pl.ds(row0 + off, sz)])
        if rem:
            @pl.when(sid == NUM_SUBCORES - 1)
            def _():
                for off in range(0, rem, CHUNK):
                    sz = min(CHUNK, rem - off)
                    pltpu.sync_copy(rows0.at[pl.ds(0, sz)],
                                    acc_sh.at[pl.ds(NUM_SUBCORES * rps + off, sz)])
        plsc.subcore_barrier()

        iwait(0, 0)
        gather(0)
        iwait(1, 1)
        gather(1)

        @pl.loop(0, nchunk // NBUF)
        def _(t):
            i0 = t * NBUF
            for b in range(NBUF):
                i = i0 + b
                gwait(b)
                scale(b, i)

                @pl.when(i + NBUF < nchunk)
                def _():
                    idxload(b, i + NBUF)
                scatter(b)
                q = (b + 2) % NBUF
                if b < 2:
                    @pl.when(t > 0)
                    def _():
                        swait(q)
                    iwait(q, i + 2)
                    gather(q)
                else:
                    swait(q)

                    @pl.when(i + 2 < nchunk)
                    def _():
                        iwait(q, i + 2)
                        gather(q)

        swait(2)
        swait(3)
        plsc.subcore_barrier()

        for off in range(0, rps, CHUNK):
            sz = min(CHUNK, rps - off)
            pltpu.sync_copy(acc_sh.at[pl.ds(row0 + off, sz)],
                            out_hbm.at[cid, pl.ds(row0 + off, sz)])
        if rem:
            @pl.when(sid == NUM_SUBCORES - 1)
            def _():
                for off in range(0, rem, CHUNK):
                    sz = min(CHUNK, rem - off)
                    r0 = NUM_SUBCORES * rps + off
                    pltpu.sync_copy(acc_sh.at[pl.ds(r0, sz)],
                                    out_hbm.at[cid, pl.ds(r0, sz)])

    return k(m, src, dst, ew)


def kernel(x, edge_index, edge_weight, weight, w_ih, w_hh, b_ih, b_hh):
    n, d = x.shape
    e = edge_index.shape[1]
    nchunk = -(-e // (NW * CHUNK))
    nchunk = -(-nchunk // 4) * 4
    e_pad = NW * nchunk * CHUNK

    src = edge_index[0].astype(jnp.int32)
    dst = edge_index[1].astype(jnp.int32)
    ew = edge_weight.astype(jnp.float32)
    pad = e_pad - e
    if pad:
        spread = jnp.arange(pad, dtype=jnp.int32) % n
        src = jnp.concatenate([src, spread])
        dst = jnp.concatenate([dst, spread])
        ew = jnp.concatenate([ew, jnp.zeros((pad,), jnp.float32)])
    src = src.reshape(NW, nchunk, CHUNK)
    dst = dst.reshape(NW, nchunk, CHUNK)
    ew = ew.reshape(NW, nchunk, CHUNK)

    wihT = w_ih.T
    cwihT = [lax.dot(weight[i], wihT, precision=lax.Precision.HIGHEST)
             for i in range(weight.shape[0])]
    whhT = w_hh.T
    bih = b_ih.reshape(1, -1)
    bhh = b_hh.reshape(1, -1)

    h = x
    for i in range(weight.shape[0]):
        gh = _tc_gh(h, whhT, bhh)
        parts = _sc_edge_agg(h, src, dst, ew, nchunk)
        h = _tc_gru(parts, h, gh, cwihT[i], bih)
    return h


# speedup vs baseline: 1.0524x; 1.0220x over previous
import functools

import jax
import jax.numpy as jnp
from jax import lax
from jax.experimental import pallas as pl
from jax.experimental.pallas import tpu as pltpu
from jax.experimental.pallas import tpu_sc as plsc

NUM_CORES = 2
NUM_SUBCORES = 16
NW = NUM_CORES * NUM_SUBCORES
LANES = 16
CHUNK = 80
NBUF = 4


def _gru_body(d, parts_ref, h_ref, wihT_ref, whhT_ref, bih_ref, bhh_ref, h_out_ref):
    agg = parts_ref[0] + parts_ref[1]
    gi = jnp.dot(agg, wihT_ref[...], preferred_element_type=jnp.float32) + bih_ref[...]
    gh = jnp.dot(h_ref[...], whhT_ref[...], preferred_element_type=jnp.float32) + bhh_ref[...]
    r = jax.nn.sigmoid(gi[:, 0:d] + gh[:, 0:d])
    z = jax.nn.sigmoid(gi[:, d:2 * d] + gh[:, d:2 * d])
    ng = jnp.tanh(gi[:, 2 * d:] + r * gh[:, 2 * d:])
    h_out_ref[...] = (1.0 - z) * ng + z * h_ref[...]


def _tc_gru(parts, h, wihT, whhT, bih, bhh):
    n, d = h.shape
    blk = 1000 if n % 1000 == 0 else n
    grid = n // blk
    return pl.pallas_call(
        functools.partial(_gru_body, d),
        grid=(grid,),
        in_specs=[pl.BlockSpec((NUM_CORES, blk, d), lambda i: (0, i, 0)),
                  pl.BlockSpec((blk, d), lambda i: (i, 0)),
                  pl.BlockSpec((d, 3 * d), lambda i: (0, 0)),
                  pl.BlockSpec((d, 3 * d), lambda i: (0, 0)),
                  pl.BlockSpec((1, 3 * d), lambda i: (0, 0)),
                  pl.BlockSpec((1, 3 * d), lambda i: (0, 0))],
        out_specs=pl.BlockSpec((blk, d), lambda i: (i, 0)),
        out_shape=jax.ShapeDtypeStruct((n, d), jnp.float32),
    )(parts, h, wihT, whhT, bih, bhh)


def _bcast_lane(vec16, j):
    idx = jnp.full((LANES, 1), j, dtype=jnp.int32)
    dnums = lax.GatherDimensionNumbers(
        offset_dims=(), collapsed_slice_dims=(0,), start_index_map=(0,))
    return lax.gather(vec16, idx, dnums, (1,),
                      mode=lax.GatherScatterMode.PROMISE_IN_BOUNDS)


def _sc_edge_agg(m, src, dst, ew, nchunk):
    n, d = m.shape
    rps = (n // NUM_SUBCORES) // 8 * 8
    rem = n - NUM_SUBCORES * rps
    mesh = plsc.VectorSubcoreMesh(core_axis_name="c", subcore_axis_name="s")

    @functools.partial(
        pl.kernel,
        out_type=jax.ShapeDtypeStruct((NUM_CORES, n, d), jnp.float32),
        mesh=mesh,
        scratch_types=[
            [pltpu.VMEM((CHUNK,), jnp.int32)] * NBUF,
            [pltpu.VMEM((CHUNK,), jnp.int32)] * NBUF,
            [pltpu.VMEM((CHUNK,), jnp.float32)] * NBUF,
            [pltpu.VMEM((CHUNK,), jnp.int32)] * NBUF,
            [pltpu.VMEM((CHUNK, d), jnp.float32)] * NBUF,
            pltpu.VMEM_SHARED((n, d), jnp.float32),
            [pltpu.SemaphoreType.DMA] * NBUF,
            [pltpu.SemaphoreType.DMA] * NBUF,
            [pltpu.SemaphoreType.DMA] * NBUF,
        ],
    )
    def k(m_hbm, src_hbm, dst_hbm, w_hbm, out_hbm,
          srcb, dstb, wb_, dbufs, rowbufs, acc_sh, gsems, ssems, isems):
        cid = lax.axis_index("c")
        sid = lax.axis_index("s")
        wid = cid * NUM_SUBCORES + sid

        def idx_copies(b, i):
            return (pltpu.make_async_copy(src_hbm.at[wid, i], srcb[b], isems[b]),
                    pltpu.make_async_copy(dst_hbm.at[wid, i], dstb[b], isems[b]),
                    pltpu.make_async_copy(w_hbm.at[wid, i], wb_[b], isems[b]))

        def idxload(b, i):
            for c in idx_copies(b, i):
                c.start()

        def iwait(b, i):
            for c in idx_copies(b, i):
                c.wait()

        def scale(b, i):
            rows, dbuf, wv, dv = rowbufs[b], dbufs[b], wb_[b], dstb[b]

            @pl.loop(0, CHUNK // LANES)
            def _(g):
                w16 = wv[pl.ds(g * LANES, LANES)]
                dbuf[pl.ds(g * LANES, LANES)] = dv[pl.ds(g * LANES, LANES)]
                for j in range(LANES):
                    wbc = _bcast_lane(w16, j)
                    e = g * LANES + j
                    for c in range(d // LANES):
                        sl = (e, pl.ds(c * LANES, LANES))
                        rows[sl] = rows[sl] * wbc

        def gather(b):
            pltpu.async_copy(m_hbm.at[srcb[b]], rowbufs[b], gsems[b])

        def gwait(b):
            pltpu.make_async_copy(m_hbm.at[srcb[b]], rowbufs[b],
                                  gsems[b]).wait()

        def scatter(b):
            pltpu.async_copy(rowbufs[b], acc_sh.at[dbufs[b]], ssems[b],
                             add=True)

        def swait(b):
            pltpu.make_async_copy(rowbufs[b], acc_sh.at[dbufs[b]],
                                  ssems[b]).wait()

        for b in range(NBUF):
            idxload(b, b)

        rows0 = rowbufs[0]

        @pl.loop(0, CHUNK)
        def _(r):
            for c in range(d // LANES):
                rows0[r, pl.ds(c * LANES, LANES)] = (
                    jnp.zeros((LANES,), jnp.float32))
        row0 = sid * rps

        def zero_copies():
            cs = [pltpu.make_async_copy(
                      rows0.at[pl.ds(0, min(CHUNK, rps - off))],
                      acc_sh.at[pl.ds(row0 + off, min(CHUNK, rps - off))],
                      ssems[0])
                  for off in range(0, rps, CHUNK)]
            return cs

        for c in zero_copies():
            c.start()
        if rem:
            @pl.when(sid == NUM_SUBCORES - 1)
            def _():
                for off in range(0, rem, CHUNK):
                    sz = min(CHUNK, rem - off)
                    pltpu.sync_copy(rows0.at[pl.ds(0, sz)],
                                    acc_sh.at[pl.ds(NUM_SUBCORES * rps + off, sz)])
        for c in zero_copies():
            c.wait()
        plsc.subcore_barrier()

        iwait(0, 0)
        gather(0)
        iwait(1, 1)
        gather(1)

        @pl.loop(0, nchunk // NBUF)
        def _(t):
            i0 = t * NBUF
            for b in range(NBUF):
                i = i0 + b
                gwait(b)
                scale(b, i)

                @pl.when(i + NBUF < nchunk)
                def _():
                    idxload(b, i + NBUF)
                scatter(b)
                q = (b + 2) % NBUF
                if b < 2:
                    @pl.when(t > 0)
                    def _():
                        swait(q)
                    iwait(q, i + 2)
                    gather(q)
                else:
                    swait(q)

                    @pl.when(i + 2 < nchunk)
                    def _():
                        iwait(q, i + 2)
                        gather(q)

        swait(2)
        swait(3)
        plsc.subcore_barrier()

        def flush_copies():
            return [pltpu.make_async_copy(
                        acc_sh.at[pl.ds(row0 + off, min(CHUNK, rps - off))],
                        out_hbm.at[cid, pl.ds(row0 + off, min(CHUNK, rps - off))],
                        ssems[1])
                    for off in range(0, rps, CHUNK)]

        for c in flush_copies():
            c.start()
        if rem:
            @pl.when(sid == NUM_SUBCORES - 1)
            def _():
                for off in range(0, rem, CHUNK):
                    sz = min(CHUNK, rem - off)
                    r0 = NUM_SUBCORES * rps + off
                    pltpu.sync_copy(acc_sh.at[pl.ds(r0, sz)],
                                    out_hbm.at[cid, pl.ds(r0, sz)])
        for c in flush_copies():
            c.wait()

    return k(m, src, dst, ew)


def kernel(x, edge_index, edge_weight, weight, w_ih, w_hh, b_ih, b_hh):
    n, d = x.shape
    e = edge_index.shape[1]
    nchunk = -(-e // (NW * CHUNK))
    nchunk = -(-nchunk // 4) * 4
    e_pad = NW * nchunk * CHUNK

    src = edge_index[0].astype(jnp.int32)
    dst = edge_index[1].astype(jnp.int32)
    ew = edge_weight.astype(jnp.float32)
    pad = e_pad - e
    if pad:
        spread = jnp.arange(pad, dtype=jnp.int32) % n
        src = jnp.concatenate([src, spread])
        dst = jnp.concatenate([dst, spread])
        ew = jnp.concatenate([ew, jnp.zeros((pad,), jnp.float32)])
    src = src.reshape(NW, nchunk, CHUNK)
    dst = dst.reshape(NW, nchunk, CHUNK)
    ew = ew.reshape(NW, nchunk, CHUNK)

    wihT = w_ih.T
    cwihT = [lax.dot(weight[i], wihT, precision=lax.Precision.HIGHEST)
             for i in range(weight.shape[0])]
    whhT = w_hh.T
    bih = b_ih.reshape(1, -1)
    bhh = b_hh.reshape(1, -1)

    h = x
    for i in range(weight.shape[0]):
        parts = _sc_edge_agg(h, src, dst, ew, nchunk)
        h = _tc_gru(parts, h, cwihT[i], whhT, bih, bhh)
    return h


# speedup vs baseline: 1.0584x; 1.0058x over previous
import functools

import jax
import jax.numpy as jnp
from jax import lax
from jax.experimental import pallas as pl
from jax.experimental.pallas import tpu as pltpu
from jax.experimental.pallas import tpu_sc as plsc

NUM_CORES = 2
NUM_SUBCORES = 16
NW = NUM_CORES * NUM_SUBCORES
LANES = 16
CHUNK = 64
NBUF = 5


def _gru_body(d, parts_ref, h_ref, wihT_ref, whhT_ref, bih_ref, bhh_ref, h_out_ref):
    agg = parts_ref[0] + parts_ref[1]
    gi = jnp.dot(agg, wihT_ref[...], preferred_element_type=jnp.float32) + bih_ref[...]
    gh = jnp.dot(h_ref[...], whhT_ref[...], preferred_element_type=jnp.float32) + bhh_ref[...]
    r = jax.nn.sigmoid(gi[:, 0:d] + gh[:, 0:d])
    z = jax.nn.sigmoid(gi[:, d:2 * d] + gh[:, d:2 * d])
    ng = jnp.tanh(gi[:, 2 * d:] + r * gh[:, 2 * d:])
    h_out_ref[...] = (1.0 - z) * ng + z * h_ref[...]


def _tc_gru(parts, h, wihT, whhT, bih, bhh):
    n, d = h.shape
    blk = 1000 if n % 1000 == 0 else n
    grid = n // blk
    return pl.pallas_call(
        functools.partial(_gru_body, d),
        grid=(grid,),
        in_specs=[pl.BlockSpec((NUM_CORES, blk, d), lambda i: (0, i, 0)),
                  pl.BlockSpec((blk, d), lambda i: (i, 0)),
                  pl.BlockSpec((d, 3 * d), lambda i: (0, 0)),
                  pl.BlockSpec((d, 3 * d), lambda i: (0, 0)),
                  pl.BlockSpec((1, 3 * d), lambda i: (0, 0)),
                  pl.BlockSpec((1, 3 * d), lambda i: (0, 0))],
        out_specs=pl.BlockSpec((blk, d), lambda i: (i, 0)),
        out_shape=jax.ShapeDtypeStruct((n, d), jnp.float32),
    )(parts, h, wihT, whhT, bih, bhh)


def _bcast_lane(vec16, j):
    idx = jnp.full((LANES, 1), j, dtype=jnp.int32)
    dnums = lax.GatherDimensionNumbers(
        offset_dims=(), collapsed_slice_dims=(0,), start_index_map=(0,))
    return lax.gather(vec16, idx, dnums, (1,),
                      mode=lax.GatherScatterMode.PROMISE_IN_BOUNDS)


def _sc_edge_agg(m, src, dst, ew, nchunk):
    n, d = m.shape
    rps = (n // NUM_SUBCORES) // 8 * 8
    rem = n - NUM_SUBCORES * rps
    mesh = plsc.VectorSubcoreMesh(core_axis_name="c", subcore_axis_name="s")

    @functools.partial(
        pl.kernel,
        out_type=jax.ShapeDtypeStruct((NUM_CORES, n, d), jnp.float32),
        mesh=mesh,
        scratch_types=[
            [pltpu.VMEM((CHUNK,), jnp.int32)] * NBUF,
            [pltpu.VMEM((CHUNK,), jnp.int32)] * NBUF,
            [pltpu.VMEM((CHUNK,), jnp.float32)] * NBUF,
            [pltpu.VMEM((CHUNK,), jnp.int32)] * NBUF,
            [pltpu.VMEM((CHUNK, d), jnp.float32)] * NBUF,
            pltpu.VMEM_SHARED((n, d), jnp.float32),
            [pltpu.SemaphoreType.DMA] * NBUF,
            [pltpu.SemaphoreType.DMA] * NBUF,
            [pltpu.SemaphoreType.DMA] * NBUF,
        ],
    )
    def k(m_hbm, src_hbm, dst_hbm, w_hbm, out_hbm,
          srcb, dstb, wb_, dbufs, rowbufs, acc_sh, gsems, ssems, isems):
        cid = lax.axis_index("c")
        sid = lax.axis_index("s")
        wid = cid * NUM_SUBCORES + sid

        def idx_copies(b, i):
            return (pltpu.make_async_copy(src_hbm.at[wid, i], srcb[b], isems[b]),
                    pltpu.make_async_copy(dst_hbm.at[wid, i], dstb[b], isems[b]),
                    pltpu.make_async_copy(w_hbm.at[wid, i], wb_[b], isems[b]))

        def idxload(b, i):
            for c in idx_copies(b, i):
                c.start()

        def iwait(b, i):
            for c in idx_copies(b, i):
                c.wait()

        def scale(b, i):
            rows, dbuf, wv, dv = rowbufs[b], dbufs[b], wb_[b], dstb[b]

            @pl.loop(0, CHUNK // LANES)
            def _(g):
                w16 = wv[pl.ds(g * LANES, LANES)]
                dbuf[pl.ds(g * LANES, LANES)] = dv[pl.ds(g * LANES, LANES)]
                for j in range(LANES):
                    wbc = _bcast_lane(w16, j)
                    e = g * LANES + j
                    for c in range(d // LANES):
                        sl = (e, pl.ds(c * LANES, LANES))
                        rows[sl] = rows[sl] * wbc

        def gather(b):
            pltpu.async_copy(m_hbm.at[srcb[b]], rowbufs[b], gsems[b])

        def gwait(b):
            pltpu.make_async_copy(m_hbm.at[srcb[b]], rowbufs[b],
                                  gsems[b]).wait()

        def scatter(b):
            pltpu.async_copy(rowbufs[b], acc_sh.at[dbufs[b]], ssems[b],
                             add=True)

        def swait(b):
            pltpu.make_async_copy(rowbufs[b], acc_sh.at[dbufs[b]],
                                  ssems[b]).wait()

        for b in range(NBUF):
            idxload(b, b)

        rows0 = rowbufs[0]

        @pl.loop(0, CHUNK)
        def _(r):
            for c in range(d // LANES):
                rows0[r, pl.ds(c * LANES, LANES)] = (
                    jnp.zeros((LANES,), jnp.float32))
        row0 = sid * rps

        def zero_copies():
            cs = [pltpu.make_async_copy(
                      rows0.at[pl.ds(0, min(CHUNK, rps - off))],
                      acc_sh.at[pl.ds(row0 + off, min(CHUNK, rps - off))],
                      ssems[0])
                  for off in range(0, rps, CHUNK)]
            return cs

        for c in zero_copies():
            c.start()
        if rem:
            @pl.when(sid == NUM_SUBCORES - 1)
            def _():
                for off in range(0, rem, CHUNK):
                    sz = min(CHUNK, rem - off)
                    pltpu.sync_copy(rows0.at[pl.ds(0, sz)],
                                    acc_sh.at[pl.ds(NUM_SUBCORES * rps + off, sz)])
        for c in zero_copies():
            c.wait()
        plsc.subcore_barrier()

        iwait(0, 0)
        gather(0)
        iwait(1, 1)
        gather(1)
        iwait(2, 2)
        gather(2)

        @pl.loop(0, nchunk // NBUF)
        def _(t):
            i0 = t * NBUF
            for b in range(NBUF):
                i = i0 + b
                gwait(b)
                scale(b, i)

                @pl.when(i + NBUF < nchunk)
                def _():
                    idxload(b, i + NBUF)
                scatter(b)
                q = (b + 3) % NBUF
                if b < 2:
                    @pl.when(t > 0)
                    def _():
                        swait(q)
                    iwait(q, i + 3)
                    gather(q)
                else:
                    swait(q)

                    @pl.when(i + 3 < nchunk)
                    def _():
                        iwait(q, i + 3)
                        gather(q)

        swait(3)
        swait(4)
        plsc.subcore_barrier()

        def flush_copies():
            return [pltpu.make_async_copy(
                        acc_sh.at[pl.ds(row0 + off, min(CHUNK, rps - off))],
                        out_hbm.at[cid, pl.ds(row0 + off, min(CHUNK, rps - off))],
                        ssems[1])
                    for off in range(0, rps, CHUNK)]

        for c in flush_copies():
            c.start()
        if rem:
            @pl.when(sid == NUM_SUBCORES - 1)
            def _():
                for off in range(0, rem, CHUNK):
                    sz = min(CHUNK, rem - off)
                    r0 = NUM_SUBCORES * rps + off
                    pltpu.sync_copy(acc_sh.at[pl.ds(r0, sz)],
                                    out_hbm.at[cid, pl.ds(r0, sz)])
        for c in flush_copies():
            c.wait()

    return k(m, src, dst, ew)


def kernel(x, edge_index, edge_weight, weight, w_ih, w_hh, b_ih, b_hh):
    n, d = x.shape
    e = edge_index.shape[1]
    nchunk = -(-e // (NW * CHUNK))
    nchunk = -(-nchunk // 4) * 4
    e_pad = NW * nchunk * CHUNK

    src = edge_index[0].astype(jnp.int32)
    dst = edge_index[1].astype(jnp.int32)
    ew = edge_weight.astype(jnp.float32)
    pad = e_pad - e
    if pad:
        spread = jnp.arange(pad, dtype=jnp.int32) % n
        src = jnp.concatenate([src, spread])
        dst = jnp.concatenate([dst, spread])
        ew = jnp.concatenate([ew, jnp.zeros((pad,), jnp.float32)])
    src = src.reshape(NW, nchunk, CHUNK)
    dst = dst.reshape(NW, nchunk, CHUNK)
    ew = ew.reshape(NW, nchunk, CHUNK)

    wihT = w_ih.T
    cwihT = [lax.dot(weight[i], wihT, precision=lax.Precision.HIGHEST)
             for i in range(weight.shape[0])]
    whhT = w_hh.T
    bih = b_ih.reshape(1, -1)
    bhh = b_hh.reshape(1, -1)

    h = x
    for i in range(weight.shape[0]):
        parts = _sc_edge_agg(h, src, dst, ew, nchunk)
        h = _tc_gru(parts, h, cwihT[i], whhT, bih, bhh)
    return h
